# Initial kernel scaffold; baseline (speedup 1.0000x reference)
import jax
import jax.numpy as jnp
from jax.experimental import pallas as pl


def kernel(x, edge_index, edge_weight, params):
    raise NotImplementedError("write your pallas kernel here")


# speedup vs baseline: 35.2877x; 35.2877x over previous
import functools

import jax
import jax.numpy as jnp
from jax import lax
from jax.experimental import pallas as pl
from jax.experimental.pallas import tpu as pltpu
from jax.experimental.pallas import tpu_sc as plsc

_N = 100000
_E = 1600000
_F = 16
_FO = 32
_HOR = 12

_NC, _NS, _L = 2, 16, 16
_NW = _NC * _NS
_NPAD = 100352
_EPAD = 1638400
_EROWS = _EPAD // 128
_ROWS_PER_W = _EROWS // _NW
_SLICE = _NPAD // _NS

_SC_PARAMS = pltpu.CompilerParams(needs_layout_passes=False,
                                  use_tc_tiling_on_sc=False)


def _deg_body(src_hbm, dst_hbm, ew_hbm, out_hbm, src_v, dst_v, ew_v, ewm_v,
              zb_v, acc_sh):
    cid = lax.axis_index("c")
    sid = lax.axis_index("s")
    base_row = (cid * _NS + sid) * _ROWS_PER_W
    slice_base = sid * _SLICE

    def _zero(i, carry):
        zb_v[pl.ds(i * _L, _L)] = jnp.zeros((_L,), jnp.float32)
        return carry
    lax.fori_loop(0, _SLICE // _L, _zero, 0)
    pltpu.sync_copy(zb_v, acc_sh.at[pl.ds(slice_base, _SLICE)])
    plsc.subcore_barrier()

    def _window(w, carry):
        r0 = base_row + w * 16
        pltpu.sync_copy(src_hbm.at[pl.ds(r0, 16)], src_v)
        pltpu.sync_copy(dst_hbm.at[pl.ds(r0, 16)], dst_v)
        pltpu.sync_copy(ew_hbm.at[pl.ds(r0, 16)], ew_v)
        for j in range(16):
            for c in range(8):
                sl = pl.ds(c * _L, _L)
                s16 = src_v[j, sl]
                d16 = dst_v[j, sl]
                e16 = ew_v[j, sl]
                ewm_v[j, sl] = jnp.where(s16 == d16, 0.0, e16)
        for j in range(16):
            pltpu.sync_copy(ewm_v.at[j], acc_sh.at[src_v.at[j]], add=True)
        return carry
    lax.fori_loop(0, _ROWS_PER_W // 16, _window, 0)

    plsc.subcore_barrier()
    pltpu.sync_copy(acc_sh.at[pl.ds(slice_base, _SLICE)],
                    out_hbm.at[cid, pl.ds(slice_base, _SLICE)])


def _run_deg(src2d, dst2d, ew2d):
    mesh = plsc.VectorSubcoreMesh(core_axis_name="c", subcore_axis_name="s")
    return pl.kernel(
        _deg_body,
        out_type=jax.ShapeDtypeStruct((_NC, _NPAD), jnp.float32),
        mesh=mesh,
        compiler_params=_SC_PARAMS,
        scratch_types=[
            pltpu.VMEM((16, 128), jnp.int32),
            pltpu.VMEM((16, 128), jnp.int32),
            pltpu.VMEM((16, 128), jnp.float32),
            pltpu.VMEM((16, 128), jnp.float32),
            pltpu.VMEM((_SLICE,), jnp.float32),
            pltpu.VMEM_SHARED((_NPAD,), jnp.float32),
        ],
    )(src2d, dst2d, ew2d)


def _dinv_body(degp_ref, out_ref):
    deg = degp_ref[0] + degp_ref[1]
    out_ref[...] = jnp.where(deg > 0, lax.rsqrt(deg), 0.0)


def _run_dinv(degp):
    out = pl.pallas_call(
        _dinv_body,
        grid=(49,),
        in_specs=[pl.BlockSpec((2, 16, 128), lambda i: (0, i, 0))],
        out_specs=pl.BlockSpec((16, 128), lambda i: (i, 0)),
        out_shape=jax.ShapeDtypeStruct((784, 128), jnp.float32),
    )(degp.reshape(_NC, 784, 128))
    return out.reshape(_NPAD)


def _prop_body(src_hbm, dst_hbm, ew_hbm, y_hbm, out_hbm,
               src_v, dst_v, ew_v, sm_v, rows_v, acc_sh, sem):
    cid = lax.axis_index("c")
    sid = lax.axis_index("s")
    base_row = (cid * _NS + sid) * _ROWS_PER_W
    slice_base = sid * _SLICE

    def _zero(i, carry):
        rows_v[i, :] = jnp.zeros((_F,), jnp.float32)
        return carry
    lax.fori_loop(0, 1024, _zero, 0)
    for p in range(6):
        pltpu.sync_copy(rows_v,
                        acc_sh.at[pl.ds(slice_base + p * 1024, 1024)])
    pltpu.sync_copy(rows_v.at[pl.ds(0, 128)],
                    acc_sh.at[pl.ds(slice_base + 6144, 128)])
    plsc.subcore_barrier()

    def _window(w, carry):
        r0 = base_row + w * 8
        pltpu.sync_copy(src_hbm.at[pl.ds(r0, 8)], src_v)
        pltpu.sync_copy(dst_hbm.at[pl.ds(r0, 8)], dst_v)
        pltpu.sync_copy(ew_hbm.at[pl.ds(r0, 8)], ew_v)
        descs = [
            pltpu.async_copy(y_hbm.at[src_v.at[j]],
                             rows_v.at[pl.ds(j * 128, 128)], sem)
            for j in range(8)
        ]
        for j in range(8):
            for c in range(8):
                sl = pl.ds(c * _L, _L)
                s16 = src_v[j, sl]
                d16 = dst_v[j, sl]
                e16 = ew_v[j, sl]
                sm_v[j, sl] = jnp.where(s16 == d16, 0.0, -e16)
        for d in descs:
            d.wait()

        for j in range(8):
            def _scale(c, carry, j=j):
                s16 = sm_v[j, pl.ds(c * _L, _L)]
                r0_ = j * 128 + c * _L
                for k in range(_L):
                    rows_v[r0_ + k, :] = rows_v[r0_ + k, :] * s16[k]
                return carry
            lax.fori_loop(0, 8, _scale, 0)

        for j in range(8):
            pltpu.sync_copy(rows_v.at[pl.ds(j * 128, 128)],
                            acc_sh.at[dst_v.at[j]], add=True)
        return carry
    lax.fori_loop(0, _ROWS_PER_W // 8, _window, 0)

    plsc.subcore_barrier()
    pltpu.sync_copy(acc_sh.at[pl.ds(slice_base, _SLICE)],
                    out_hbm.at[cid, pl.ds(slice_base, _SLICE)])


def _run_prop(src2d, dst2d, ew2d, y2):
    mesh = plsc.VectorSubcoreMesh(core_axis_name="c", subcore_axis_name="s")
    return pl.kernel(
        _prop_body,
        out_type=jax.ShapeDtypeStruct((_NC, _NPAD, _F), jnp.float32),
        mesh=mesh,
        compiler_params=_SC_PARAMS,
        scratch_types=[
            pltpu.VMEM((8, 128), jnp.int32),
            pltpu.VMEM((8, 128), jnp.int32),
            pltpu.VMEM((8, 128), jnp.float32),
            pltpu.VMEM((8, 128), jnp.float32),
            pltpu.VMEM((1024, _F), jnp.float32),
            pltpu.VMEM_SHARED((_NPAD, _F), jnp.float32),
            pltpu.SemaphoreType.DMA,
        ],
    )(src2d, dst2d, ew2d, y2)


def _gates_body(x_ref, ap_ref, dr_ref, w0i_ref, w1i_ref, bi_ref, w0c_ref,
                w1c_ref, bc_ref, w0o_ref, w1o_ref, bo_ref, wco_ref, wl_ref,
                bl_ref, h_ref, hh_ref, cc_ref):
    x = x_ref[...]
    px = dr_ref[...] * (ap_ref[0] + ap_ref[1])
    f32 = jnp.float32
    zi = (jnp.dot(x, w0i_ref[...], preferred_element_type=f32)
          + jnp.dot(px, w1i_ref[...], preferred_element_type=f32)
          + bi_ref[...])
    zc = (jnp.dot(x, w0c_ref[...], preferred_element_type=f32)
          + jnp.dot(px, w1c_ref[...], preferred_element_type=f32)
          + bc_ref[...])
    zo = (jnp.dot(x, w0o_ref[...], preferred_element_type=f32)
          + jnp.dot(px, w1o_ref[...], preferred_element_type=f32)
          + bo_ref[...])
    gi = jax.nn.sigmoid(zi)
    gt = jnp.tanh(zc)
    c = gi * gt
    go = jax.nn.sigmoid(zo + wco_ref[...] * c)
    hh = go * jnp.tanh(c)
    h_ref[...] = (jnp.dot(jnp.maximum(hh, 0.0), wl_ref[...],
                          preferred_element_type=f32) + bl_ref[...])
    hh_ref[...] = hh
    cc_ref[...] = c


def _run_gates(x2, ap, dinvR, params):
    p = params
    w0i, w1i = p["Wx0_i"], p["Wx1_i"]
    w0c, w1c = p["Wx0_c"], p["Wx1_c"]
    w0o, w1o = p["Wx0_o"], p["Wx1_o"]
    bi = (p["bx_i"] + p["bh_i"] + p["b_i"][0]).reshape(1, _FO)
    bc = (p["bx_c"] + p["bh_c"] + p["b_c"][0]).reshape(1, _FO)
    bo = (p["bx_o"] + p["bh_o"] + p["b_o"][0]).reshape(1, _FO)
    wco = p["w_c_o"].reshape(1, _FO)
    wl = p["W_lin"]
    bl = p["b_lin"].reshape(1, _HOR)

    R = 512
    grid = (_NPAD // R,)
    full = lambda shape: pl.BlockSpec(shape, lambda i: tuple(0 for _ in shape))
    return pl.pallas_call(
        _gates_body,
        grid=grid,
        in_specs=[
            pl.BlockSpec((R, _F), lambda i: (i, 0)),
            pl.BlockSpec((2, R, _F), lambda i: (0, i, 0)),
            pl.BlockSpec((R, _F), lambda i: (i, 0)),
            full((_F, _FO)), full((_F, _FO)), full((1, _FO)),
            full((_F, _FO)), full((_F, _FO)), full((1, _FO)),
            full((_F, _FO)), full((_F, _FO)), full((1, _FO)),
            full((1, _FO)), full((_FO, _HOR)), full((1, _HOR)),
        ],
        out_specs=[
            pl.BlockSpec((R, _HOR), lambda i: (i, 0)),
            pl.BlockSpec((R, _FO), lambda i: (i, 0)),
            pl.BlockSpec((R, _FO), lambda i: (i, 0)),
        ],
        out_shape=[
            jax.ShapeDtypeStruct((_NPAD, _HOR), jnp.float32),
            jax.ShapeDtypeStruct((_NPAD, _FO), jnp.float32),
            jax.ShapeDtypeStruct((_NPAD, _FO), jnp.float32),
        ],
    )(x2, ap, dinvR, w0i, w1i, bi, w0c, w1c, bc, w0o, w1o, bo, wco, wl, bl)


def kernel(x, edge_index, edge_weight, params):
    x2 = jnp.pad(jnp.squeeze(x, axis=1), ((0, _NPAD - _N), (0, 0)))
    pad = _EPAD - _E
    padidx = jnp.arange(pad, dtype=jnp.int32) % _N
    src = jnp.concatenate([edge_index[0], padidx]).reshape(_EROWS, 128)
    dst = jnp.concatenate([edge_index[1], padidx]).reshape(_EROWS, 128)
    ew = jnp.concatenate(
        [edge_weight, jnp.zeros((pad,), jnp.float32)]).reshape(_EROWS, 128)

    degp = _run_deg(src, dst, ew)
    dinv = _run_dinv(degp)
    dinvR = jnp.broadcast_to(dinv[:, None], (_NPAD, _F))
    y2 = dinvR * x2
    ap = _run_prop(src, dst, ew, y2)
    h, hh, cc = _run_gates(x2, ap, dinvR, params)
    return (h[:_N], hh[:_N], cc[:_N])


# speedup vs baseline: 42.0004x; 1.1902x over previous
import jax
import jax.numpy as jnp
from jax import lax
from jax.experimental import pallas as pl
from jax.experimental.pallas import tpu as pltpu
from jax.experimental.pallas import tpu_sc as plsc

_N = 100000
_E = 1600000
_F = 16
_FO = 32
_HOR = 12

_NC, _NS, _L = 2, 16, 16
_NW = _NC * _NS
_NPAD = 100352
_EROWS = _E // 128
_WROWS = _EROWS // _NW
_XTRA = _EROWS - _WROWS * _NW
_SLICE = _NPAD // _NS

_SC_PARAMS = pltpu.CompilerParams(needs_layout_passes=False,
                                  use_tc_tiling_on_sc=False)


def _deg_body(src_hbm, dst_hbm, ew_hbm, out_hbm, src_v, dst_v, ew_v, ewm_v,
              zb_v, acc_sh):
    cid = lax.axis_index("c")
    sid = lax.axis_index("s")
    wid = cid * _NS + sid
    base_row = wid * _WROWS
    slice_base = sid * _SLICE

    def _zero(i, carry):
        zb_v[pl.ds(i * _L, _L)] = jnp.zeros((_L,), jnp.float32)
        return carry
    lax.fori_loop(0, _SLICE // _L, _zero, 0)
    pltpu.sync_copy(zb_v, acc_sh.at[pl.ds(slice_base, _SLICE)])
    plsc.subcore_barrier()

    def _mask_row(jrow, src_ref, dst_ref, ew_ref, out_ref):
        for c in range(8):
            sl = pl.ds(c * _L, _L)
            s16 = src_ref[jrow, sl]
            d16 = dst_ref[jrow, sl]
            e16 = ew_ref[jrow, sl]
            out_ref[jrow, sl] = jnp.where(s16 == d16, 0.0, e16)

    def _window(w, carry):
        r0 = base_row + w * 15
        pltpu.sync_copy(src_hbm.at[pl.ds(r0, 15)], src_v)
        pltpu.sync_copy(dst_hbm.at[pl.ds(r0, 15)], dst_v)
        pltpu.sync_copy(ew_hbm.at[pl.ds(r0, 15)], ew_v)
        for j in range(15):
            _mask_row(j, src_v, dst_v, ew_v, ewm_v)
        for j in range(15):
            pltpu.sync_copy(ewm_v.at[j], acc_sh.at[src_v.at[j]], add=True)
        return carry
    lax.fori_loop(0, _WROWS // 15, _window, 0)

    @pl.when(wid < _XTRA)
    def _tail():
        r = _NW * _WROWS + wid
        pltpu.sync_copy(src_hbm.at[r], src_v.at[0])
        pltpu.sync_copy(dst_hbm.at[r], dst_v.at[0])
        pltpu.sync_copy(ew_hbm.at[r], ew_v.at[0])
        _mask_row(0, src_v, dst_v, ew_v, ewm_v)
        pltpu.sync_copy(ewm_v.at[0], acc_sh.at[src_v.at[0]], add=True)

    plsc.subcore_barrier()
    pltpu.sync_copy(acc_sh.at[pl.ds(slice_base, _SLICE)],
                    out_hbm.at[cid, pl.ds(slice_base, _SLICE)])


def _run_deg(src2d, dst2d, ew2d):
    mesh = plsc.VectorSubcoreMesh(core_axis_name="c", subcore_axis_name="s")
    return pl.kernel(
        _deg_body,
        out_type=jax.ShapeDtypeStruct((_NC, _NPAD), jnp.float32),
        mesh=mesh,
        compiler_params=_SC_PARAMS,
        scratch_types=[
            pltpu.VMEM((15, 128), jnp.int32),
            pltpu.VMEM((15, 128), jnp.int32),
            pltpu.VMEM((15, 128), jnp.float32),
            pltpu.VMEM((15, 128), jnp.float32),
            pltpu.VMEM((_SLICE,), jnp.float32),
            pltpu.VMEM_SHARED((_NPAD,), jnp.float32),
        ],
    )(src2d, dst2d, ew2d)


def _dinv_body(degp_ref, out_ref):
    deg = degp_ref[0] + degp_ref[1]
    out_ref[...] = jnp.where(deg > 0, lax.rsqrt(deg), 0.0)


def _run_dinv(degp):
    out = pl.pallas_call(
        _dinv_body,
        grid=(49,),
        in_specs=[pl.BlockSpec((2, 16, 128), lambda i: (0, i, 0))],
        out_specs=pl.BlockSpec((16, 128), lambda i: (i, 0)),
        out_shape=jax.ShapeDtypeStruct((784, 128), jnp.float32),
    )(degp.reshape(_NC, 784, 128))
    return out.reshape(_NPAD)


def _prop_body(src_hbm, dst_hbm, ew_hbm, y_hbm, out_hbm,
               src_v, dst_v, ew_v, sm_v, rows_v, acc_sh, sem):
    cid = lax.axis_index("c")
    sid = lax.axis_index("s")
    wid = cid * _NS + sid
    base_row = wid * _WROWS
    slice_base = sid * _SLICE

    def _zero(i, carry):
        rows_v[i, :] = jnp.zeros((_F,), jnp.float32)
        return carry
    lax.fori_loop(0, 768, _zero, 0)
    for p in range(8):
        pltpu.sync_copy(rows_v, acc_sh.at[pl.ds(slice_base + p * 768, 768)])
    pltpu.sync_copy(rows_v.at[pl.ds(0, 128)],
                    acc_sh.at[pl.ds(slice_base + 6144, 128)])
    plsc.subcore_barrier()

    def _do_rows(nrows):
        descs = []
        for j in range(nrows):
            descs.append(pltpu.async_copy(
                y_hbm.at[src_v.at[j]], rows_v.at[pl.ds(j * 128, 128)], sem))
        for j in range(nrows):
            for c in range(8):
                sl = pl.ds(c * _L, _L)
                s16 = src_v[j, sl]
                d16 = dst_v[j, sl]
                e16 = ew_v[j, sl]
                sm_v[j, sl] = jnp.where(s16 == d16, 0.0, -e16)
        for d in descs:
            d.wait()
        for j in range(nrows):
            def _scale(c, carry, j=j):
                s16 = sm_v[j, pl.ds(c * _L, _L)]
                r0_ = j * 128 + c * _L
                for k in range(_L):
                    rows_v[r0_ + k, :] = rows_v[r0_ + k, :] * s16[k]
                return carry
            lax.fori_loop(0, 8, _scale, 0)
        for j in range(nrows):
            pltpu.sync_copy(rows_v.at[pl.ds(j * 128, 128)],
                            acc_sh.at[dst_v.at[j]], add=True)

    def _window(w, carry):
        r0 = base_row + w * 6
        pltpu.sync_copy(src_hbm.at[pl.ds(r0, 6)], src_v)
        pltpu.sync_copy(dst_hbm.at[pl.ds(r0, 6)], dst_v)
        pltpu.sync_copy(ew_hbm.at[pl.ds(r0, 6)], ew_v)
        _do_rows(6)
        return carry
    lax.fori_loop(0, _WROWS // 6, _window, 0)

    @pl.when(wid < _XTRA)
    def _tail():
        r = _NW * _WROWS + wid
        pltpu.sync_copy(src_hbm.at[r], src_v.at[0])
        pltpu.sync_copy(dst_hbm.at[r], dst_v.at[0])
        pltpu.sync_copy(ew_hbm.at[r], ew_v.at[0])
        _do_rows(1)

    plsc.subcore_barrier()
    pltpu.sync_copy(acc_sh.at[pl.ds(sid * (_N // _NS), _N // _NS)],
                    out_hbm.at[cid, pl.ds(sid * (_N // _NS), _N // _NS)])


def _run_prop(src2d, dst2d, ew2d, y2):
    mesh = plsc.VectorSubcoreMesh(core_axis_name="c", subcore_axis_name="s")
    return pl.kernel(
        _prop_body,
        out_type=jax.ShapeDtypeStruct((_NC, _N, _F), jnp.float32),
        mesh=mesh,
        compiler_params=_SC_PARAMS,
        scratch_types=[
            pltpu.VMEM((6, 128), jnp.int32),
            pltpu.VMEM((6, 128), jnp.int32),
            pltpu.VMEM((6, 128), jnp.float32),
            pltpu.VMEM((6, 128), jnp.float32),
            pltpu.VMEM((768, _F), jnp.float32),
            pltpu.VMEM_SHARED((_NPAD, _F), jnp.float32),
            pltpu.SemaphoreType.DMA,
        ],
    )(src2d, dst2d, ew2d, y2)


def _gates_body(x_ref, ap_ref, dv_ref, w0i_ref, w1i_ref, bi_ref, w0c_ref,
                w1c_ref, bc_ref, w0o_ref, w1o_ref, bo_ref, wco_ref, wl_ref,
                bl_ref, h_ref, hh_ref, cc_ref):
    x = x_ref[...]
    px = dv_ref[...] * (ap_ref[0] + ap_ref[1])
    f32 = jnp.float32
    zi = (jnp.dot(x, w0i_ref[...], preferred_element_type=f32)
          + jnp.dot(px, w1i_ref[...], preferred_element_type=f32)
          + bi_ref[...])
    zc = (jnp.dot(x, w0c_ref[...], preferred_element_type=f32)
          + jnp.dot(px, w1c_ref[...], preferred_element_type=f32)
          + bc_ref[...])
    zo = (jnp.dot(x, w0o_ref[...], preferred_element_type=f32)
          + jnp.dot(px, w1o_ref[...], preferred_element_type=f32)
          + bo_ref[...])
    gi = jax.nn.sigmoid(zi)
    gt = jnp.tanh(zc)
    c = gi * gt
    go = jax.nn.sigmoid(zo + wco_ref[...] * c)
    hh = go * jnp.tanh(c)
    h_ref[...] = (jnp.dot(jnp.maximum(hh, 0.0), wl_ref[...],
                          preferred_element_type=f32) + bl_ref[...])
    hh_ref[...] = hh
    cc_ref[...] = c


def _run_gates(x2, ap, dinv1, params):
    p = params
    bi = (p["bx_i"] + p["bh_i"] + p["b_i"][0]).reshape(1, _FO)
    bc = (p["bx_c"] + p["bh_c"] + p["b_c"][0]).reshape(1, _FO)
    bo = (p["bx_o"] + p["bh_o"] + p["b_o"][0]).reshape(1, _FO)
    wco = p["w_c_o"].reshape(1, _FO)
    bl = p["b_lin"].reshape(1, _HOR)

    R = 2000
    grid = (_N // R,)
    full = lambda shape: pl.BlockSpec(shape, lambda i: tuple(0 for _ in shape))
    return pl.pallas_call(
        _gates_body,
        grid=grid,
        in_specs=[
            pl.BlockSpec((R, _F), lambda i: (i, 0)),
            pl.BlockSpec((2, R, _F), lambda i: (0, i, 0)),
            pl.BlockSpec((R, 1), lambda i: (i, 0)),
            full((_F, _FO)), full((_F, _FO)), full((1, _FO)),
            full((_F, _FO)), full((_F, _FO)), full((1, _FO)),
            full((_F, _FO)), full((_F, _FO)), full((1, _FO)),
            full((1, _FO)), full((_FO, _HOR)), full((1, _HOR)),
        ],
        out_specs=[
            pl.BlockSpec((R, _HOR), lambda i: (i, 0)),
            pl.BlockSpec((R, _FO), lambda i: (i, 0)),
            pl.BlockSpec((R, _FO), lambda i: (i, 0)),
        ],
        out_shape=[
            jax.ShapeDtypeStruct((_N, _HOR), jnp.float32),
            jax.ShapeDtypeStruct((_N, _FO), jnp.float32),
            jax.ShapeDtypeStruct((_N, _FO), jnp.float32),
        ],
    )(x2, ap, dinv1, p["Wx0_i"], p["Wx1_i"], bi, p["Wx0_c"], p["Wx1_c"], bc,
      p["Wx0_o"], p["Wx1_o"], bo, wco, p["W_lin"], bl)


def kernel(x, edge_index, edge_weight, params):
    x2 = jnp.squeeze(x, axis=1)
    src = edge_index[0].reshape(_EROWS, 128)
    dst = edge_index[1].reshape(_EROWS, 128)
    ew = edge_weight.reshape(_EROWS, 128)

    degp = _run_deg(src, dst, ew)
    dinv = _run_dinv(degp)
    dinv1 = dinv[:_N, None]
    y2 = dinv1 * x2
    ap = _run_prop(src, dst, ew, y2)
    h, hh, cc = _run_gates(x2, ap, dinv1, params)
    return (h, hh, cc)


# speedup vs baseline: 44.7950x; 1.0665x over previous
import jax
import jax.numpy as jnp
from jax import lax
from jax.experimental import pallas as pl
from jax.experimental.pallas import tpu as pltpu
from jax.experimental.pallas import tpu_sc as plsc

_N = 100000
_E = 1600000
_F = 16
_FO = 32
_HOR = 12

_NC, _NS, _L = 2, 16, 16
_NW = _NC * _NS
_NPAD = 100352
_EROWS = _E // 128
_WROWS = _EROWS // _NW
_XTRA = _EROWS - _WROWS * _NW
_SLICE = _NPAD // _NS

_SC_PARAMS = pltpu.CompilerParams(needs_layout_passes=False,
                                  use_tc_tiling_on_sc=False)


def _deg_body(src_hbm, dst_hbm, ew_hbm, out_hbm, src_v, dst_v, ew_v, ewm_v,
              zb_v, acc_sh, sem):
    cid = lax.axis_index("c")
    sid = lax.axis_index("s")
    wid = cid * _NS + sid
    base_row = wid * _WROWS
    slice_base = sid * _SLICE

    def _zero(i, carry):
        zb_v[pl.ds(i * _L, _L)] = jnp.zeros((_L,), jnp.float32)
        return carry
    lax.fori_loop(0, _SLICE // _L, _zero, 0)
    pltpu.sync_copy(zb_v, acc_sh.at[pl.ds(slice_base, _SLICE)])
    plsc.subcore_barrier()

    def _mask_row(jrow, src_ref, dst_ref, ew_ref, out_ref):
        for c in range(8):
            sl = pl.ds(c * _L, _L)
            s16 = src_ref[jrow, sl]
            d16 = dst_ref[jrow, sl]
            e16 = ew_ref[jrow, sl]
            out_ref[jrow, sl] = jnp.where(s16 == d16, 0.0, e16)

    def _window(w, carry):
        r0 = base_row + w * 15
        pltpu.sync_copy(src_hbm.at[pl.ds(r0, 15)], src_v)
        pltpu.sync_copy(dst_hbm.at[pl.ds(r0, 15)], dst_v)
        pltpu.sync_copy(ew_hbm.at[pl.ds(r0, 15)], ew_v)
        for j in range(15):
            _mask_row(j, src_v, dst_v, ew_v, ewm_v)
        descs = [pltpu.async_copy(ewm_v.at[j], acc_sh.at[src_v.at[j]],
                                  sem, add=True) for j in range(15)]
        for d in descs:
            d.wait()
        return carry
    lax.fori_loop(0, _WROWS // 15, _window, 0)

    @pl.when(wid < _XTRA)
    def _tail():
        r = _NW * _WROWS + wid
        pltpu.sync_copy(src_hbm.at[r], src_v.at[0])
        pltpu.sync_copy(dst_hbm.at[r], dst_v.at[0])
        pltpu.sync_copy(ew_hbm.at[r], ew_v.at[0])
        _mask_row(0, src_v, dst_v, ew_v, ewm_v)
        pltpu.sync_copy(ewm_v.at[0], acc_sh.at[src_v.at[0]], add=True)

    plsc.subcore_barrier()
    pltpu.sync_copy(acc_sh.at[pl.ds(slice_base, _SLICE)],
                    out_hbm.at[cid, pl.ds(slice_base, _SLICE)])


def _run_deg(src2d, dst2d, ew2d):
    mesh = plsc.VectorSubcoreMesh(core_axis_name="c", subcore_axis_name="s")
    return pl.kernel(
        _deg_body,
        out_type=jax.ShapeDtypeStruct((_NC, _NPAD), jnp.float32),
        mesh=mesh,
        compiler_params=_SC_PARAMS,
        scratch_types=[
            pltpu.VMEM((15, 128), jnp.int32),
            pltpu.VMEM((15, 128), jnp.int32),
            pltpu.VMEM((15, 128), jnp.float32),
            pltpu.VMEM((15, 128), jnp.float32),
            pltpu.VMEM((_SLICE,), jnp.float32),
            pltpu.VMEM_SHARED((_NPAD,), jnp.float32),
            pltpu.SemaphoreType.DMA,
        ],
    )(src2d, dst2d, ew2d)


def _dinv_body(degp_ref, out_ref):
    deg = degp_ref[0] + degp_ref[1]
    out_ref[...] = jnp.where(deg > 0, lax.rsqrt(deg), 0.0)


def _run_dinv(degp):
    out = pl.pallas_call(
        _dinv_body,
        grid=(49,),
        in_specs=[pl.BlockSpec((2, 16, 128), lambda i: (0, i, 0))],
        out_specs=pl.BlockSpec((16, 128), lambda i: (i, 0)),
        out_shape=jax.ShapeDtypeStruct((784, 128), jnp.float32),
    )(degp.reshape(_NC, 784, 128))
    return out.reshape(_NPAD)


def _prop_body(src_hbm, dst_hbm, ew_hbm, y_hbm, dinv_hbm, out_hbm,
               src_v, dst_v, ew_v, sm_v, rows_v, dvv, acc_sh, sem, sem2):
    cid = lax.axis_index("c")
    sid = lax.axis_index("s")
    wid = cid * _NS + sid
    base_row = wid * _WROWS
    slice_base = sid * _SLICE

    pltpu.sync_copy(dinv_hbm.at[pl.ds(slice_base, _SLICE)], dvv)

    def _zero(i, carry):
        rows_v[i, :] = jnp.zeros((_F,), jnp.float32)
        return carry
    lax.fori_loop(0, 768, _zero, 0)
    for p in range(8):
        pltpu.sync_copy(rows_v, acc_sh.at[pl.ds(slice_base + p * 768, 768)])
    pltpu.sync_copy(rows_v.at[pl.ds(0, 128)],
                    acc_sh.at[pl.ds(slice_base + 6144, 128)])
    plsc.subcore_barrier()

    def _do_rows(nrows):
        descs = []
        for j in range(nrows):
            descs.append(pltpu.async_copy(
                y_hbm.at[src_v.at[j]], rows_v.at[pl.ds(j * 128, 128)], sem))
        for j in range(nrows):
            for c in range(8):
                sl = pl.ds(c * _L, _L)
                s16 = src_v[j, sl]
                d16 = dst_v[j, sl]
                e16 = ew_v[j, sl]
                sm_v[j, sl] = jnp.where(s16 == d16, 0.0, -e16)
        for d in descs:
            d.wait()
        for j in range(nrows):
            def _scale(c, carry, j=j):
                s16 = sm_v[j, pl.ds(c * _L, _L)]
                r0_ = j * 128 + c * _L
                for k in range(_L):
                    rows_v[r0_ + k, :] = rows_v[r0_ + k, :] * s16[k]
                return carry
            lax.fori_loop(0, 8, _scale, 0)
        sdescs = [pltpu.async_copy(rows_v.at[pl.ds(j * 128, 128)],
                                   acc_sh.at[dst_v.at[j]], sem2, add=True)
                  for j in range(nrows)]
        for d in sdescs:
            d.wait()

    def _window(w, carry):
        r0 = base_row + w * 6
        pltpu.sync_copy(src_hbm.at[pl.ds(r0, 6)], src_v)
        pltpu.sync_copy(dst_hbm.at[pl.ds(r0, 6)], dst_v)
        pltpu.sync_copy(ew_hbm.at[pl.ds(r0, 6)], ew_v)
        _do_rows(6)
        return carry
    lax.fori_loop(0, _WROWS // 6, _window, 0)

    @pl.when(wid < _XTRA)
    def _tail():
        r = _NW * _WROWS + wid
        pltpu.sync_copy(src_hbm.at[r], src_v.at[0])
        pltpu.sync_copy(dst_hbm.at[r], dst_v.at[0])
        pltpu.sync_copy(ew_hbm.at[r], ew_v.at[0])
        _do_rows(1)

    plsc.subcore_barrier()
    def _drain_chunk(off, n):
        pltpu.sync_copy(acc_sh.at[pl.ds(slice_base + off, n)],
                        rows_v.at[pl.ds(0, n)])

        def _dscale(c, carry):
            dv16 = dvv[pl.ds(off + c * _L, _L)]
            for k in range(_L):
                rows_v[c * _L + k, :] = rows_v[c * _L + k, :] * dv16[k]
            return carry
        lax.fori_loop(0, n // _L, _dscale, 0)
        pltpu.sync_copy(rows_v.at[pl.ds(0, n)],
                        out_hbm.at[cid, pl.ds(slice_base + off, n)])
    for p in range(8):
        _drain_chunk(p * 768, 768)
    _drain_chunk(6144, 128)


def _run_prop(src2d, dst2d, ew2d, y2, dinv):
    mesh = plsc.VectorSubcoreMesh(core_axis_name="c", subcore_axis_name="s")
    return pl.kernel(
        _prop_body,
        out_type=jax.ShapeDtypeStruct((_NC, _NPAD, _F), jnp.float32),
        mesh=mesh,
        compiler_params=_SC_PARAMS,
        scratch_types=[
            pltpu.VMEM((6, 128), jnp.int32),
            pltpu.VMEM((6, 128), jnp.int32),
            pltpu.VMEM((6, 128), jnp.float32),
            pltpu.VMEM((6, 128), jnp.float32),
            pltpu.VMEM((768, _F), jnp.float32),
            pltpu.VMEM((_SLICE,), jnp.float32),
            pltpu.VMEM_SHARED((_NPAD, _F), jnp.float32),
            pltpu.SemaphoreType.DMA,
            pltpu.SemaphoreType.DMA,
        ],
    )(src2d, dst2d, ew2d, y2, dinv)


def _gates_body(x_ref, ap_ref, w0i_ref, w1i_ref, bi_ref, w0c_ref,
                w1c_ref, bc_ref, w0o_ref, w1o_ref, bo_ref, wco_ref, wl_ref,
                bl_ref, h_ref, hh_ref, cc_ref):
    x = x_ref[...]
    px = ap_ref[0] + ap_ref[1]
    f32 = jnp.float32
    zi = (jnp.dot(x, w0i_ref[...], preferred_element_type=f32)
          + jnp.dot(px, w1i_ref[...], preferred_element_type=f32)
          + bi_ref[...])
    zc = (jnp.dot(x, w0c_ref[...], preferred_element_type=f32)
          + jnp.dot(px, w1c_ref[...], preferred_element_type=f32)
          + bc_ref[...])
    zo = (jnp.dot(x, w0o_ref[...], preferred_element_type=f32)
          + jnp.dot(px, w1o_ref[...], preferred_element_type=f32)
          + bo_ref[...])
    gi = jax.nn.sigmoid(zi)
    gt = jnp.tanh(zc)
    c = gi * gt
    go = jax.nn.sigmoid(zo + wco_ref[...] * c)
    hh = go * jnp.tanh(c)
    h_ref[...] = (jnp.dot(jnp.maximum(hh, 0.0), wl_ref[...],
                          preferred_element_type=f32) + bl_ref[...])
    hh_ref[...] = hh
    cc_ref[...] = c


def _run_gates(x2, ap, params):
    p = params
    bi = (p["bx_i"] + p["bh_i"] + p["b_i"][0]).reshape(1, _FO)
    bc = (p["bx_c"] + p["bh_c"] + p["b_c"][0]).reshape(1, _FO)
    bo = (p["bx_o"] + p["bh_o"] + p["b_o"][0]).reshape(1, _FO)
    wco = p["w_c_o"].reshape(1, _FO)
    bl = p["b_lin"].reshape(1, _HOR)

    R = 2000
    grid = (_N // R,)
    full = lambda shape: pl.BlockSpec(shape, lambda i: tuple(0 for _ in shape))
    return pl.pallas_call(
        _gates_body,
        grid=grid,
        in_specs=[
            pl.BlockSpec((R, _F), lambda i: (i, 0)),
            pl.BlockSpec((2, R, _F), lambda i: (0, i, 0)),
            full((_F, _FO)), full((_F, _FO)), full((1, _FO)),
            full((_F, _FO)), full((_F, _FO)), full((1, _FO)),
            full((_F, _FO)), full((_F, _FO)), full((1, _FO)),
            full((1, _FO)), full((_FO, _HOR)), full((1, _HOR)),
        ],
        out_specs=[
            pl.BlockSpec((R, _HOR), lambda i: (i, 0)),
            pl.BlockSpec((R, _FO), lambda i: (i, 0)),
            pl.BlockSpec((R, _FO), lambda i: (i, 0)),
        ],
        out_shape=[
            jax.ShapeDtypeStruct((_N, _HOR), jnp.float32),
            jax.ShapeDtypeStruct((_N, _FO), jnp.float32),
            jax.ShapeDtypeStruct((_N, _FO), jnp.float32),
        ],
    )(x2, ap, p["Wx0_i"], p["Wx1_i"], bi, p["Wx0_c"], p["Wx1_c"], bc,
      p["Wx0_o"], p["Wx1_o"], bo, wco, p["W_lin"], bl)


def kernel(x, edge_index, edge_weight, params):
    x2 = jnp.squeeze(x, axis=1)
    src = edge_index[0].reshape(_EROWS, 128)
    dst = edge_index[1].reshape(_EROWS, 128)
    ew = edge_weight.reshape(_EROWS, 128)

    degp = _run_deg(src, dst, ew)
    dinv = _run_dinv(degp)
    y2 = dinv[:_N, None] * x2
    ap = _run_prop(src, dst, ew, y2, dinv)
    h, hh, cc = _run_gates(x2, ap, params)
    return (h, hh, cc)


# speedup vs baseline: 48.3849x; 1.0801x over previous
import jax
import jax.numpy as jnp
from jax import lax
from jax.experimental import pallas as pl
from jax.experimental.pallas import tpu as pltpu
from jax.experimental.pallas import tpu_sc as plsc

_N = 100000
_E = 1600000
_F = 16
_FO = 32
_HOR = 12

_NC, _NS, _L = 2, 16, 16
_NW = _NC * _NS
_NPAD = 100352
_EROWS = _E // 128
_WROWS = _EROWS // _NW
_XTRA = _EROWS - _WROWS * _NW
_SLICE = _NPAD // _NS

_SC_PARAMS = pltpu.CompilerParams(needs_layout_passes=False,
                                  use_tc_tiling_on_sc=False)


def _deg_body(src_hbm, dst_hbm, ew_hbm, out_hbm, src_v, dst_v, ew_v, ewm_v,
              zb_v, acc_sh, sem):
    cid = lax.axis_index("c")
    sid = lax.axis_index("s")
    wid = cid * _NS + sid
    base_row = wid * _WROWS
    slice_base = sid * _SLICE

    def _zero(i, carry):
        zb_v[pl.ds(i * _L, _L)] = jnp.zeros((_L,), jnp.float32)
        return carry
    lax.fori_loop(0, _SLICE // _L, _zero, 0)
    pltpu.sync_copy(zb_v, acc_sh.at[pl.ds(slice_base, _SLICE)])
    plsc.subcore_barrier()

    def _mask_row(jrow, src_ref, dst_ref, ew_ref, out_ref):
        for c in range(8):
            sl = pl.ds(c * _L, _L)
            s16 = src_ref[jrow, sl]
            d16 = dst_ref[jrow, sl]
            e16 = ew_ref[jrow, sl]
            out_ref[jrow, sl] = jnp.where(s16 == d16, 0.0, e16)

    def _window(w, carry):
        r0 = base_row + w * 15
        pltpu.sync_copy(src_hbm.at[pl.ds(r0, 15)], src_v)
        pltpu.sync_copy(dst_hbm.at[pl.ds(r0, 15)], dst_v)
        pltpu.sync_copy(ew_hbm.at[pl.ds(r0, 15)], ew_v)
        for j in range(15):
            _mask_row(j, src_v, dst_v, ew_v, ewm_v)
        descs = [pltpu.async_copy(ewm_v.at[j], acc_sh.at[src_v.at[j]],
                                  sem, add=True) for j in range(15)]
        for d in descs:
            d.wait()
        return carry
    lax.fori_loop(0, _WROWS // 15, _window, 0)

    @pl.when(wid < _XTRA)
    def _tail():
        r = _NW * _WROWS + wid
        pltpu.sync_copy(src_hbm.at[r], src_v.at[0])
        pltpu.sync_copy(dst_hbm.at[r], dst_v.at[0])
        pltpu.sync_copy(ew_hbm.at[r], ew_v.at[0])
        _mask_row(0, src_v, dst_v, ew_v, ewm_v)
        pltpu.sync_copy(ewm_v.at[0], acc_sh.at[src_v.at[0]], add=True)

    plsc.subcore_barrier()
    pltpu.sync_copy(acc_sh.at[pl.ds(slice_base, _SLICE)],
                    out_hbm.at[cid, pl.ds(slice_base, _SLICE)])


def _run_deg(src2d, dst2d, ew2d):
    mesh = plsc.VectorSubcoreMesh(core_axis_name="c", subcore_axis_name="s")
    return pl.kernel(
        _deg_body,
        out_type=jax.ShapeDtypeStruct((_NC, _NPAD), jnp.float32),
        mesh=mesh,
        compiler_params=_SC_PARAMS,
        scratch_types=[
            pltpu.VMEM((15, 128), jnp.int32),
            pltpu.VMEM((15, 128), jnp.int32),
            pltpu.VMEM((15, 128), jnp.float32),
            pltpu.VMEM((15, 128), jnp.float32),
            pltpu.VMEM((_SLICE,), jnp.float32),
            pltpu.VMEM_SHARED((_NPAD,), jnp.float32),
            pltpu.SemaphoreType.DMA,
        ],
    )(src2d, dst2d, ew2d)


def _dinv_body(degp_ref, out_ref):
    deg = degp_ref[0] + degp_ref[1]
    out_ref[...] = jnp.where(deg > 0, lax.rsqrt(deg), 0.0)


def _run_dinv(degp):
    out = pl.pallas_call(
        _dinv_body,
        grid=(49,),
        in_specs=[pl.BlockSpec((2, 16, 128), lambda i: (0, i, 0))],
        out_specs=pl.BlockSpec((16, 128), lambda i: (i, 0)),
        out_shape=jax.ShapeDtypeStruct((784, 128), jnp.float32),
    )(degp.reshape(_NC, 784, 128))
    return out.reshape(_NPAD)


_WIN = 5
_NWIN = _WROWS // _WIN


def _prop_body(src_hbm, dst_hbm, ew_hbm, y_hbm, dinv_hbm, out_hbm,
               src0, dst0, ew0, sm0, rows0, src1, dst1, ew1, sm1, rows1,
               dvc, acc_sh, semg0, semg1, sems):
    cid = lax.axis_index("c")
    sid = lax.axis_index("s")
    wid = cid * _NS + sid
    base_row = wid * _WROWS
    slice_base = sid * _SLICE
    bufs = ((src0, dst0, ew0, sm0, rows0, semg0),
            (src1, dst1, ew1, sm1, rows1, semg1))

    def _zero(i, carry):
        rows0[i, :] = jnp.zeros((_F,), jnp.float32)
        return carry
    lax.fori_loop(0, 640, _zero, 0)
    for p in range(9):
        pltpu.sync_copy(rows0, acc_sh.at[pl.ds(slice_base + p * 640, 640)])
    pltpu.sync_copy(rows0.at[pl.ds(0, 512)],
                    acc_sh.at[pl.ds(slice_base + 5760, 512)])
    plsc.subcore_barrier()

    def _load_idx(b, r0):
        src_v, dst_v, ew_v = bufs[b][0], bufs[b][1], bufs[b][2]
        pltpu.sync_copy(src_hbm.at[pl.ds(r0, _WIN)], src_v)
        pltpu.sync_copy(dst_hbm.at[pl.ds(r0, _WIN)], dst_v)
        pltpu.sync_copy(ew_hbm.at[pl.ds(r0, _WIN)], ew_v)

    def _fire_gathers(b):
        src_v, rows_v, semg = bufs[b][0], bufs[b][4], bufs[b][5]
        return [pltpu.async_copy(y_hbm.at[src_v.at[j]],
                                 rows_v.at[pl.ds(j * 128, 128)], semg)
                for j in range(_WIN)]

    def _process(b, nrows=_WIN):
        src_v, dst_v, ew_v, sm_v, rows_v, semg = bufs[b]
        for j in range(nrows):
            for c in range(8):
                sl = pl.ds(c * _L, _L)
                sm_v[j, sl] = jnp.where(src_v[j, sl] == dst_v[j, sl],
                                        0.0, -ew_v[j, sl])
        for d in _fire_drain_list(b, nrows):
            d.wait()
        for j in range(nrows):
            def _scale(c, carry, j=j):
                s16 = sm_v[j, pl.ds(c * _L, _L)]
                r0_ = j * 128 + c * _L
                for k in range(_L):
                    rows_v[r0_ + k, :] = rows_v[r0_ + k, :] * s16[k]
                return carry
            lax.fori_loop(0, 8, _scale, 0)
        sdescs = [pltpu.async_copy(rows_v.at[pl.ds(j * 128, 128)],
                                   acc_sh.at[dst_v.at[j]], sems, add=True)
                  for j in range(nrows)]
        for d in sdescs:
            d.wait()

    def _fire_drain_list(b, nrows):
        src_v, rows_v, semg = bufs[b][0], bufs[b][4], bufs[b][5]
        return [pltpu.make_async_copy(y_hbm.at[src_v.at[j]],
                                      rows_v.at[pl.ds(j * 128, 128)], semg)
                for j in range(nrows)]

    _load_idx(0, base_row)
    _fire_gathers(0)

    def _step(k, carry):
        r1 = base_row + (2 * k + 1) * _WIN
        _load_idx(1, r1)
        _fire_gathers(1)
        _process(0)

        @pl.when(k < _NWIN // 2 - 1)
        def _prefetch():
            r2 = base_row + (2 * k + 2) * _WIN
            _load_idx(0, r2)
            _fire_gathers(0)
        _process(1)
        return carry
    lax.fori_loop(0, _NWIN // 2, _step, 0)

    @pl.when(wid < _XTRA)
    def _tail():
        r = _NW * _WROWS + wid
        pltpu.sync_copy(src_hbm.at[r], src0.at[0])
        pltpu.sync_copy(dst_hbm.at[r], dst0.at[0])
        pltpu.sync_copy(ew_hbm.at[r], ew0.at[0])
        pltpu.async_copy(y_hbm.at[src0.at[0]], rows0.at[pl.ds(0, 128)], semg0)
        _process(0, nrows=1)

    plsc.subcore_barrier()
    def _drain_chunk(off, n):
        pltpu.sync_copy(dinv_hbm.at[pl.ds(slice_base + off, n)],
                        dvc.at[pl.ds(0, n)])
        pltpu.sync_copy(acc_sh.at[pl.ds(slice_base + off, n)],
                        rows0.at[pl.ds(0, n)])

        def _dscale(c, carry):
            dv16 = dvc[pl.ds(c * _L, _L)]
            for k in range(_L):
                rows0[c * _L + k, :] = rows0[c * _L + k, :] * dv16[k]
            return carry
        lax.fori_loop(0, n // _L, _dscale, 0)
        pltpu.sync_copy(rows0.at[pl.ds(0, n)],
                        out_hbm.at[cid, pl.ds(slice_base + off, n)])
    for p in range(9):
        _drain_chunk(p * 640, 640)
    _drain_chunk(5760, 512)


def _run_prop(src2d, dst2d, ew2d, y2, dinv):
    mesh = plsc.VectorSubcoreMesh(core_axis_name="c", subcore_axis_name="s")
    return pl.kernel(
        _prop_body,
        out_type=jax.ShapeDtypeStruct((_NC, _NPAD, _F), jnp.float32),
        mesh=mesh,
        compiler_params=_SC_PARAMS,
        scratch_types=[
            pltpu.VMEM((_WIN, 128), jnp.int32),
            pltpu.VMEM((_WIN, 128), jnp.int32),
            pltpu.VMEM((_WIN, 128), jnp.float32),
            pltpu.VMEM((_WIN, 128), jnp.float32),
            pltpu.VMEM((_WIN * 128, _F), jnp.float32),
            pltpu.VMEM((_WIN, 128), jnp.int32),
            pltpu.VMEM((_WIN, 128), jnp.int32),
            pltpu.VMEM((_WIN, 128), jnp.float32),
            pltpu.VMEM((_WIN, 128), jnp.float32),
            pltpu.VMEM((_WIN * 128, _F), jnp.float32),
            pltpu.VMEM((_WIN * 128,), jnp.float32),
            pltpu.VMEM_SHARED((_NPAD, _F), jnp.float32),
            pltpu.SemaphoreType.DMA,
            pltpu.SemaphoreType.DMA,
            pltpu.SemaphoreType.DMA,
        ],
    )(src2d, dst2d, ew2d, y2, dinv)


def _gates_body(x_ref, ap_ref, w0i_ref, w1i_ref, bi_ref, w0c_ref,
                w1c_ref, bc_ref, w0o_ref, w1o_ref, bo_ref, wco_ref, wl_ref,
                bl_ref, h_ref, hh_ref, cc_ref):
    x = x_ref[...]
    px = ap_ref[0] + ap_ref[1]
    f32 = jnp.float32
    zi = (jnp.dot(x, w0i_ref[...], preferred_element_type=f32)
          + jnp.dot(px, w1i_ref[...], preferred_element_type=f32)
          + bi_ref[...])
    zc = (jnp.dot(x, w0c_ref[...], preferred_element_type=f32)
          + jnp.dot(px, w1c_ref[...], preferred_element_type=f32)
          + bc_ref[...])
    zo = (jnp.dot(x, w0o_ref[...], preferred_element_type=f32)
          + jnp.dot(px, w1o_ref[...], preferred_element_type=f32)
          + bo_ref[...])
    gi = jax.nn.sigmoid(zi)
    gt = jnp.tanh(zc)
    c = gi * gt
    go = jax.nn.sigmoid(zo + wco_ref[...] * c)
    hh = go * jnp.tanh(c)
    h_ref[...] = (jnp.dot(jnp.maximum(hh, 0.0), wl_ref[...],
                          preferred_element_type=f32) + bl_ref[...])
    hh_ref[...] = hh
    cc_ref[...] = c


def _run_gates(x2, ap, params):
    p = params
    bi = (p["bx_i"] + p["bh_i"] + p["b_i"][0]).reshape(1, _FO)
    bc = (p["bx_c"] + p["bh_c"] + p["b_c"][0]).reshape(1, _FO)
    bo = (p["bx_o"] + p["bh_o"] + p["b_o"][0]).reshape(1, _FO)
    wco = p["w_c_o"].reshape(1, _FO)
    bl = p["b_lin"].reshape(1, _HOR)

    R = 2000
    grid = (_N // R,)
    full = lambda shape: pl.BlockSpec(shape, lambda i: tuple(0 for _ in shape))
    return pl.pallas_call(
        _gates_body,
        grid=grid,
        in_specs=[
            pl.BlockSpec((R, _F), lambda i: (i, 0)),
            pl.BlockSpec((2, R, _F), lambda i: (0, i, 0)),
            full((_F, _FO)), full((_F, _FO)), full((1, _FO)),
            full((_F, _FO)), full((_F, _FO)), full((1, _FO)),
            full((_F, _FO)), full((_F, _FO)), full((1, _FO)),
            full((1, _FO)), full((_FO, _HOR)), full((1, _HOR)),
        ],
        out_specs=[
            pl.BlockSpec((R, _HOR), lambda i: (i, 0)),
            pl.BlockSpec((R, _FO), lambda i: (i, 0)),
            pl.BlockSpec((R, _FO), lambda i: (i, 0)),
        ],
        out_shape=[
            jax.ShapeDtypeStruct((_N, _HOR), jnp.float32),
            jax.ShapeDtypeStruct((_N, _FO), jnp.float32),
            jax.ShapeDtypeStruct((_N, _FO), jnp.float32),
        ],
    )(x2, ap, p["Wx0_i"], p["Wx1_i"], bi, p["Wx0_c"], p["Wx1_c"], bc,
      p["Wx0_o"], p["Wx1_o"], bo, wco, p["W_lin"], bl)


def kernel(x, edge_index, edge_weight, params):
    x2 = jnp.squeeze(x, axis=1)
    src = edge_index[0].reshape(_EROWS, 128)
    dst = edge_index[1].reshape(_EROWS, 128)
    ew = edge_weight.reshape(_EROWS, 128)

    degp = _run_deg(src, dst, ew)
    dinv = _run_dinv(degp)
    y2 = dinv[:_N, None] * x2
    ap = _run_prop(src, dst, ew, y2, dinv)
    h, hh, cc = _run_gates(x2, ap, params)
    return (h, hh, cc)
